# Initial kernel scaffold; baseline (speedup 1.0000x reference)
import jax
import jax.numpy as jnp
from jax.experimental import pallas as pl


def kernel(inputs, weight):
    raise NotImplementedError("write your pallas kernel here")


# speedup vs baseline: 1.1230x; 1.1230x over previous
import functools

import jax
import jax.numpy as jnp
from jax import lax
from jax.experimental import pallas as pl
from jax.experimental.pallas import tpu as pltpu
from jax.experimental.pallas import tpu_sc as plsc

NUM_EMB = 1_000_000
DIM = 32
SCALE = 1e-3

NUM_CORES = 2
NUM_SUBCORES = 16
NUM_WORKERS = NUM_CORES * NUM_SUBCORES
LANES = 16

B_TOTAL = 16384 * 50
PER_W = B_TOTAL // NUM_WORKERS
CHUNK = 1024
NCHUNK = PER_W // CHUNK
UNROLL = 8


def _sc_body(idx_hbm, table_hbm, out_hbm, idx_v, rows_v, sem):
    wid = lax.axis_index("s") * NUM_CORES + lax.axis_index("c")
    base = wid * PER_W
    scale_vec = jnp.full((LANES,), SCALE, dtype=jnp.float32)

    def chunk_body(c, carry):
        off = base + c * CHUNK
        pltpu.sync_copy(idx_hbm.at[pl.ds(off, CHUNK)], idx_v)
        pltpu.async_copy(table_hbm.at[idx_v], rows_v, sem).wait()

        def scale_body(i, carry2):
            r = i * UNROLL
            for u in range(UNROLL):
                rows_v[r + u, pl.ds(0, LANES)] = (
                    rows_v[r + u, pl.ds(0, LANES)] * scale_vec)
                rows_v[r + u, pl.ds(LANES, LANES)] = (
                    rows_v[r + u, pl.ds(LANES, LANES)] * scale_vec)
            return carry2

        lax.fori_loop(0, CHUNK // UNROLL, scale_body, 0, unroll=False)
        pltpu.sync_copy(rows_v, out_hbm.at[pl.ds(off, CHUNK)])
        return carry

    lax.fori_loop(0, NCHUNK, chunk_body, 0, unroll=False)


@functools.partial(
    pl.kernel,
    out_type=jax.ShapeDtypeStruct((B_TOTAL, DIM), jnp.float32),
    mesh=plsc.VectorSubcoreMesh(core_axis_name="c", subcore_axis_name="s"),
    scratch_types=[
        pltpu.VMEM((CHUNK,), jnp.int32),
        pltpu.VMEM((CHUNK, DIM), jnp.float32),
        pltpu.SemaphoreType.DMA,
    ],
    compiler_params=pltpu.CompilerParams(use_tc_tiling_on_sc=False),
)
def _gather_scaled(idx_hbm, table_hbm, out_hbm, idx_v, rows_v, sem):
    _sc_body(idx_hbm, table_hbm, out_hbm, idx_v, rows_v, sem)


def kernel(inputs, weight):
    idx_flat = inputs.reshape(-1)
    out = _gather_scaled(idx_flat, weight)
    return out.reshape(inputs.shape + (DIM,))


# speedup vs baseline: 1.1594x; 1.0324x over previous
import functools

import jax
import jax.numpy as jnp
from jax import lax
from jax.experimental import pallas as pl
from jax.experimental.pallas import tpu as pltpu
from jax.experimental.pallas import tpu_sc as plsc

NUM_EMB = 1_000_000
DIM = 32
SCALE = 1e-3

NUM_CORES = 2
NUM_SUBCORES = 16
NUM_WORKERS = NUM_CORES * NUM_SUBCORES
LANES = 16

B_TOTAL = 16384 * 50
PER_W = B_TOTAL // NUM_WORKERS
CHUNK = 1024
NCHUNK = PER_W // CHUNK
UNROLL = 8


def _sc_body(idx_hbm, table_hbm, out_hbm, idx_all, rows, sg0, sg1, ss0, ss1):
    wid = lax.axis_index("s") * NUM_CORES + lax.axis_index("c")
    base = wid * PER_W
    scale_vec = jnp.full((LANES,), SCALE, dtype=jnp.float32)
    sg = (sg0, sg1)
    ss = (ss0, ss1)

    pltpu.sync_copy(idx_hbm.at[wid], idx_all)

    def scale_chunk(b):
        def scale_body(i, carry):
            r = i * UNROLL
            for u in range(UNROLL):
                rows[b, r + u, pl.ds(0, LANES)] = (
                    rows[b, r + u, pl.ds(0, LANES)] * scale_vec)
                rows[b, r + u, pl.ds(LANES, LANES)] = (
                    rows[b, r + u, pl.ds(LANES, LANES)] * scale_vec)
            return carry
        lax.fori_loop(0, CHUNK // UNROLL, scale_body, 0, unroll=False)

    gd, sd = {}, {}
    gd[0] = pltpu.async_copy(table_hbm.at[idx_all.at[0]], rows.at[0], sg[0])
    for c in range(NCHUNK):
        b = c & 1
        nb = 1 - b
        if c + 1 < NCHUNK:
            if c >= 1:
                sd[c - 1].wait()
            gd[c + 1] = pltpu.async_copy(
                table_hbm.at[idx_all.at[c + 1]], rows.at[nb], sg[nb])
        gd[c].wait()
        scale_chunk(b)
        sd[c] = pltpu.async_copy(
            rows.at[b], out_hbm.at[pl.ds(base + c * CHUNK, CHUNK)], ss[b])
    sd[NCHUNK - 2].wait()
    sd[NCHUNK - 1].wait()


@functools.partial(
    pl.kernel,
    out_type=jax.ShapeDtypeStruct((B_TOTAL, DIM), jnp.float32),
    mesh=plsc.VectorSubcoreMesh(core_axis_name="c", subcore_axis_name="s"),
    scratch_types=[
        pltpu.VMEM((NCHUNK, CHUNK), jnp.int32),
        pltpu.VMEM((2, CHUNK, DIM), jnp.float32),
        pltpu.SemaphoreType.DMA,
        pltpu.SemaphoreType.DMA,
        pltpu.SemaphoreType.DMA,
        pltpu.SemaphoreType.DMA,
    ],
    compiler_params=pltpu.CompilerParams(use_tc_tiling_on_sc=False),
)
def _gather_scaled(idx_hbm, table_hbm, out_hbm, idx_all, rows,
                   sg0, sg1, ss0, ss1):
    _sc_body(idx_hbm, table_hbm, out_hbm, idx_all, rows, sg0, sg1, ss0, ss1)


def kernel(inputs, weight):
    idx_resh = inputs.reshape(NUM_WORKERS, NCHUNK, CHUNK)
    out = _gather_scaled(idx_resh, weight)
    return out.reshape(inputs.shape + (DIM,))


# speedup vs baseline: 1.1596x; 1.0002x over previous
import functools

import jax
import jax.numpy as jnp
from jax import lax
from jax.experimental import pallas as pl
from jax.experimental.pallas import tpu as pltpu
from jax.experimental.pallas import tpu_sc as plsc

NUM_EMB = 1_000_000
DIM = 32
SCALE = 1e-3

NUM_CORES = 2
NUM_SUBCORES = 16
NUM_WORKERS = NUM_CORES * NUM_SUBCORES
LANES = 16

B_TOTAL = 16384 * 50
PER_W = B_TOTAL // NUM_WORKERS
CHUNK = 1024
NCHUNK = PER_W // CHUNK
UNROLL = 8


def _sc_body(idx_hbm, table_hbm, out_hbm, idx_all, rows, sg0, sg1, ss0, ss1):
    wid = lax.axis_index("s") * NUM_CORES + lax.axis_index("c")
    base = wid * PER_W
    scale_vec = jnp.full((LANES,), SCALE, dtype=jnp.float32)
    sg = (sg0, sg1)
    ss = (ss0, ss1)

    pltpu.sync_copy(idx_hbm.at[wid], idx_all)

    def scale_chunk(b):
        def scale_body(i, carry):
            r = i * UNROLL
            for u in range(UNROLL):
                rows[b, r + u, pl.ds(0, LANES)] = (
                    rows[b, r + u, pl.ds(0, LANES)] * scale_vec)
                rows[b, r + u, pl.ds(LANES, LANES)] = (
                    rows[b, r + u, pl.ds(LANES, LANES)] * scale_vec)
            return carry
        lax.fori_loop(0, CHUNK // UNROLL, scale_body, 0, unroll=False)

    gd, sd = {}, {}
    gd[0] = pltpu.async_copy(table_hbm.at[idx_all.at[0]], rows.at[0], sg[0])
    for c in range(NCHUNK):
        b = c & 1
        nb = 1 - b
        if c + 1 < NCHUNK:
            if c >= 1:
                sd[c - 1].wait()
            gd[c + 1] = pltpu.async_copy(
                table_hbm.at[idx_all.at[c + 1]], rows.at[nb], sg[nb])
        gd[c].wait()
        scale_chunk(b)
        sd[c] = pltpu.async_copy(
            rows.at[b], out_hbm.at[pl.ds(base + c * CHUNK, CHUNK)], ss[b])
    sd[NCHUNK - 2].wait()
    sd[NCHUNK - 1].wait()


@functools.partial(
    pl.kernel,
    out_type=jax.ShapeDtypeStruct((B_TOTAL, DIM), jnp.float32),
    mesh=plsc.VectorSubcoreMesh(core_axis_name="c", subcore_axis_name="s"),
    scratch_types=[
        pltpu.VMEM((NCHUNK, CHUNK), jnp.int32),
        pltpu.VMEM((2, CHUNK, DIM), jnp.float32),
        pltpu.SemaphoreType.DMA,
        pltpu.SemaphoreType.DMA,
        pltpu.SemaphoreType.DMA,
        pltpu.SemaphoreType.DMA,
    ],
    compiler_params=pltpu.CompilerParams(
        use_tc_tiling_on_sc=False, skip_device_barrier=True),
)
def _gather_scaled(idx_hbm, table_hbm, out_hbm, idx_all, rows,
                   sg0, sg1, ss0, ss1):
    _sc_body(idx_hbm, table_hbm, out_hbm, idx_all, rows, sg0, sg1, ss0, ss1)


def kernel(inputs, weight):
    idx_resh = inputs.reshape(NUM_WORKERS, NCHUNK, CHUNK)
    out = _gather_scaled(idx_resh, weight)
    return out.reshape(inputs.shape + (DIM,))


# speedup vs baseline: 1.6296x; 1.4053x over previous
import functools

import jax
import jax.numpy as jnp
from jax import lax
from jax.experimental import pallas as pl
from jax.experimental.pallas import tpu as pltpu
from jax.experimental.pallas import tpu_sc as plsc

NUM_EMB = 1_000_000
DIM = 32
SCALE = 1e-3

NUM_WORKERS = 32
LANES = 16

NP = 50
NS = 16384
UNIT = 512
NSUB = UNIT // 128
NUNITS = NP * NS // UNIT
PER_W = NUNITS // NUM_WORKERS
SB_PER_P = NS // UNIT


def _sc_body(idx_hbm, table_hbm, out_hbm, idx_all, rows, tbuf,
             sg0, sg1, sw0, sw1):
    wid = lax.axis_index("s") * 2 + lax.axis_index("c")
    u0 = wid * PER_W
    scale_vec = jnp.full((LANES,), SCALE, dtype=jnp.float32)
    sg = (sg0, sg1)
    sw = (sw0, sw1)

    pltpu.sync_copy(idx_hbm.at[pl.ds(wid * PER_W * NSUB, PER_W * NSUB)],
                    idx_all)

    def start_gathers(k, b):
        return [
            pltpu.async_copy(
                table_hbm.at[idx_all.at[NSUB * k + q]], rows.at[b, q], sg[b])
            for q in range(NSUB)
        ]

    def compute(b):
        def dbody(d, carry):
            ti = d // 8
            rr = d % 8
            dvec = jnp.full((LANES,), d, dtype=jnp.int32)
            for q in range(NSUB):
                qvec = jnp.full((LANES,), q, dtype=jnp.int32)
                for cb in range(8):
                    ridx = jnp.arange(LANES, dtype=jnp.int32) + (cb * LANES)
                    v = plsc.load_gather(rows.at[b], [qvec, ridx, dvec])
                    tbuf[b, ti, pl.ds(q * 1024 + rr * 128 + cb * LANES,
                                      LANES)] = v * scale_vec
            return carry
        lax.fori_loop(0, DIM, dbody, 0, unroll=False)

    def wait_gathers(k, b):
        for q in range(NSUB):
            pltpu.make_async_copy(
                table_hbm.at[idx_all.at[NSUB * k + q]], rows.at[b, q],
                sg[b]).wait()

    def wait_write(b):
        pltpu.make_async_copy(
            tbuf.at[b], out_hbm.at[0, :, pl.ds(0, UNIT * 8)], sw[b]).wait()

    def start_write(k, b):
        u = u0 + k
        p = u // SB_PER_P
        sb = u % SB_PER_P
        pltpu.async_copy(
            tbuf.at[b], out_hbm.at[p, :, pl.ds(sb * (UNIT * 8), UNIT * 8)],
            sw[b])

    start_gathers(0, 0)
    start_gathers(1, 1)
    for b in range(2):
        wait_gathers(b, b)
        compute(b)
        start_write(b, b)
        start_gathers(b + 2, b)

    def body(kk, carry):
        for b in range(2):
            k = 2 * kk + b
            wait_gathers(k, b)
            wait_write(b)
            compute(b)
            start_write(k, b)
            start_gathers(k + 2, b)
        return carry
    lax.fori_loop(1, PER_W // 2 - 1, body, 0, unroll=False)

    for b in range(2):
        k = PER_W - 2 + b
        wait_gathers(k, b)
        wait_write(b)
        compute(b)
        start_write(k, b)
    for b in range(2):
        wait_write(b)


@functools.partial(
    pl.kernel,
    out_type=jax.ShapeDtypeStruct((NP, DIM // 8, NS * 8), jnp.float32),
    mesh=plsc.VectorSubcoreMesh(core_axis_name="c", subcore_axis_name="s"),
    scratch_types=[
        pltpu.VMEM((PER_W * NSUB, 128), jnp.int32),
        pltpu.VMEM((2, NSUB, 128, DIM), jnp.float32),
        pltpu.VMEM((2, DIM // 8, UNIT * 8), jnp.float32),
        pltpu.SemaphoreType.DMA,
        pltpu.SemaphoreType.DMA,
        pltpu.SemaphoreType.DMA,
        pltpu.SemaphoreType.DMA,
    ],
    compiler_params=pltpu.CompilerParams(
        use_tc_tiling_on_sc=False, needs_layout_passes=False),
)
def _gather_scaled(idx_hbm, table_hbm, out_hbm, idx_all, rows, tbuf,
                   sg0, sg1, sw0, sw1):
    _sc_body(idx_hbm, table_hbm, out_hbm, idx_all, rows, tbuf,
             sg0, sg1, sw0, sw1)


def kernel(inputs, weight):
    idx5 = inputs.T.reshape(NUNITS * NSUB, 128)
    p_out = _gather_scaled(idx5, weight)
    out = (p_out.reshape(NP, DIM // 8, NS // 128, 8, 128)
           .transpose(2, 4, 0, 1, 3)
           .reshape(NS, NP, DIM))
    return out


# speedup vs baseline: 1.8917x; 1.1609x over previous
import functools

import jax
import jax.numpy as jnp
from jax import lax
from jax.experimental import pallas as pl
from jax.experimental.pallas import tpu as pltpu
from jax.experimental.pallas import tpu_sc as plsc

NUM_EMB = 1_000_000
DIM = 32
SCALE = 1e-3

NUM_WORKERS = 32
LANES = 16

NP = 50
NS = 16384
UNIT = 512
NSUB = UNIT // 128
NUNITS = NP * NS // UNIT
PER_W = NUNITS // NUM_WORKERS
SB_PER_P = NS // UNIT


def _sc_body(idx_hbm, table_hbm, out_hbm, idx_all, rows, tbuf,
             sg0, sg1, sw0, sw1):
    wid = lax.axis_index("s") * 2 + lax.axis_index("c")
    u0 = wid * PER_W
    scale_vec = jnp.full((LANES,), SCALE, dtype=jnp.float32)
    sg = (sg0, sg1)
    sw = (sw0, sw1)

    pltpu.sync_copy(idx_hbm.at[pl.ds(wid * PER_W * NSUB, PER_W * NSUB)],
                    idx_all)

    def start_gathers(k, b):
        return [
            pltpu.async_copy(
                table_hbm.at[idx_all.at[NSUB * k + q]], rows.at[b, q], sg[b])
            for q in range(NSUB)
        ]

    dvec = lax.iota(jnp.int32, LANES)
    ivec_lo = (dvec // 8) * 4096 + (dvec % 8) * 128
    ivec_hi = ((dvec + LANES) // 8) * 4096 + ((dvec + LANES) % 8) * 128

    def compute(b):
        def ibody(i, carry):
            for q in range(NSUB):
                bvec = jnp.full((LANES,), q * 1024 + i, dtype=jnp.int32)
                lo = rows[b, q, i, pl.ds(0, LANES)] * scale_vec
                hi = rows[b, q, i, pl.ds(LANES, LANES)] * scale_vec
                plsc.store_scatter(tbuf.at[b], [ivec_lo + bvec], lo)
                plsc.store_scatter(tbuf.at[b], [ivec_hi + bvec], hi)
            return carry
        lax.fori_loop(0, 128, ibody, 0, unroll=False)

    def wait_gathers(k, b):
        for q in range(NSUB):
            pltpu.make_async_copy(
                table_hbm.at[idx_all.at[NSUB * k + q]], rows.at[b, q],
                sg[b]).wait()

    def wait_write(b):
        for ti in range(DIM // 8):
            pltpu.make_async_copy(
                tbuf.at[b, pl.ds(ti * (UNIT * 8), UNIT * 8)],
                out_hbm.at[0, ti, pl.ds(0, UNIT * 8)], sw[b]).wait()

    def start_write(k, b):
        u = u0 + k
        p = u // SB_PER_P
        sb = u % SB_PER_P
        for ti in range(DIM // 8):
            pltpu.async_copy(
                tbuf.at[b, pl.ds(ti * (UNIT * 8), UNIT * 8)],
                out_hbm.at[p, ti, pl.ds(sb * (UNIT * 8), UNIT * 8)],
                sw[b])

    start_gathers(0, 0)
    start_gathers(1, 1)
    for b in range(2):
        wait_gathers(b, b)
        compute(b)
        start_write(b, b)
        start_gathers(b + 2, b)

    def body(kk, carry):
        for b in range(2):
            k = 2 * kk + b
            wait_gathers(k, b)
            wait_write(b)
            compute(b)
            start_write(k, b)
            start_gathers(k + 2, b)
        return carry
    lax.fori_loop(1, PER_W // 2 - 1, body, 0, unroll=False)

    for b in range(2):
        k = PER_W - 2 + b
        wait_gathers(k, b)
        wait_write(b)
        compute(b)
        start_write(k, b)
    for b in range(2):
        wait_write(b)


@functools.partial(
    pl.kernel,
    out_type=jax.ShapeDtypeStruct((NP, DIM // 8, NS * 8), jnp.float32),
    mesh=plsc.VectorSubcoreMesh(core_axis_name="c", subcore_axis_name="s"),
    scratch_types=[
        pltpu.VMEM((PER_W * NSUB, 128), jnp.int32),
        pltpu.VMEM((2, NSUB, 128, DIM), jnp.float32),
        pltpu.VMEM((2, (DIM // 8) * UNIT * 8), jnp.float32),
        pltpu.SemaphoreType.DMA,
        pltpu.SemaphoreType.DMA,
        pltpu.SemaphoreType.DMA,
        pltpu.SemaphoreType.DMA,
    ],
    compiler_params=pltpu.CompilerParams(
        use_tc_tiling_on_sc=False, needs_layout_passes=False),
)
def _gather_scaled(idx_hbm, table_hbm, out_hbm, idx_all, rows, tbuf,
                   sg0, sg1, sw0, sw1):
    _sc_body(idx_hbm, table_hbm, out_hbm, idx_all, rows, tbuf,
             sg0, sg1, sw0, sw1)


def kernel(inputs, weight):
    idx5 = inputs.T.reshape(NUNITS * NSUB, 128)
    p_out = _gather_scaled(idx5, weight)
    out = (p_out.reshape(NP, DIM // 8, NS // 128, 8, 128)
           .transpose(2, 4, 0, 1, 3)
           .reshape(NS, NP, DIM))
    return out


# speedup vs baseline: 2.1160x; 1.1185x over previous
import functools

import jax
import jax.numpy as jnp
from jax import lax
from jax.experimental import pallas as pl
from jax.experimental.pallas import tpu as pltpu
from jax.experimental.pallas import tpu_sc as plsc

NUM_EMB = 1_000_000
DIM = 32
SCALE = 1e-3

NUM_WORKERS = 32
LANES = 16

NP = 50
NS = 16384
UNIT = 512
NSUB = UNIT // 128
NUNITS = NP * NS // UNIT
PER_W = NUNITS // NUM_WORKERS
SB_PER_P = NS // UNIT


def _sc_body(idx_hbm, table_hbm, out_hbm, idx_all, rows, tbuf,
             sg0, sg1, sw0, sw1):
    wid = lax.axis_index("s") * 2 + lax.axis_index("c")
    u0 = wid * PER_W
    scale_vec = jnp.full((LANES,), SCALE, dtype=jnp.float32)
    sg = (sg0, sg1)
    sw = (sw0, sw1)

    pltpu.sync_copy(idx_hbm.at[pl.ds(wid * PER_W * NSUB, PER_W * NSUB)],
                    idx_all)

    def start_gathers(k, b):
        return [
            pltpu.async_copy(
                table_hbm.at[idx_all.at[NSUB * k + q]], rows.at[b, q], sg[b])
            for q in range(NSUB)
        ]

    dvec = lax.iota(jnp.int32, LANES)
    ivec_lo = (dvec // 8) * 4096 + (dvec % 8) * 128
    ivec_hi = ((dvec + LANES) // 8) * 4096 + ((dvec + LANES) % 8) * 128

    def compute(b):
        @plsc.parallel_loop(0, 128, 1, unroll=4)
        def ibody(i):
            for q in range(NSUB):
                bvec = jnp.full((LANES,), q * 1024 + i, dtype=jnp.int32)
                lo = rows[b, q, i, pl.ds(0, LANES)] * scale_vec
                hi = rows[b, q, i, pl.ds(LANES, LANES)] * scale_vec
                plsc.store_scatter(tbuf.at[b], [ivec_lo + bvec], lo)
                plsc.store_scatter(tbuf.at[b], [ivec_hi + bvec], hi)

    def wait_gathers(k, b):
        for q in range(NSUB):
            pltpu.make_async_copy(
                table_hbm.at[idx_all.at[NSUB * k + q]], rows.at[b, q],
                sg[b]).wait()

    def wait_write(b):
        for ti in range(DIM // 8):
            pltpu.make_async_copy(
                tbuf.at[b, pl.ds(ti * (UNIT * 8), UNIT * 8)],
                out_hbm.at[0, ti, pl.ds(0, UNIT * 8)], sw[b]).wait()

    def start_write(k, b):
        u = u0 + k
        p = u // SB_PER_P
        sb = u % SB_PER_P
        for ti in range(DIM // 8):
            pltpu.async_copy(
                tbuf.at[b, pl.ds(ti * (UNIT * 8), UNIT * 8)],
                out_hbm.at[p, ti, pl.ds(sb * (UNIT * 8), UNIT * 8)],
                sw[b])

    start_gathers(0, 0)
    start_gathers(1, 1)
    for b in range(2):
        wait_gathers(b, b)
        compute(b)
        start_write(b, b)
        start_gathers(b + 2, b)

    def body(kk, carry):
        for b in range(2):
            k = 2 * kk + b
            wait_gathers(k, b)
            wait_write(b)
            compute(b)
            start_write(k, b)
            start_gathers(k + 2, b)
        return carry
    lax.fori_loop(1, PER_W // 2 - 1, body, 0, unroll=False)

    for b in range(2):
        k = PER_W - 2 + b
        wait_gathers(k, b)
        wait_write(b)
        compute(b)
        start_write(k, b)
    for b in range(2):
        wait_write(b)


@functools.partial(
    pl.kernel,
    out_type=jax.ShapeDtypeStruct((NP, DIM // 8, NS * 8), jnp.float32),
    mesh=plsc.VectorSubcoreMesh(core_axis_name="c", subcore_axis_name="s"),
    scratch_types=[
        pltpu.VMEM((PER_W * NSUB, 128), jnp.int32),
        pltpu.VMEM((2, NSUB, 128, DIM), jnp.float32),
        pltpu.VMEM((2, (DIM // 8) * UNIT * 8), jnp.float32),
        pltpu.SemaphoreType.DMA,
        pltpu.SemaphoreType.DMA,
        pltpu.SemaphoreType.DMA,
        pltpu.SemaphoreType.DMA,
    ],
    compiler_params=pltpu.CompilerParams(
        use_tc_tiling_on_sc=False, needs_layout_passes=False),
)
def _gather_scaled(idx_hbm, table_hbm, out_hbm, idx_all, rows, tbuf,
                   sg0, sg1, sw0, sw1):
    _sc_body(idx_hbm, table_hbm, out_hbm, idx_all, rows, tbuf,
             sg0, sg1, sw0, sw1)


def kernel(inputs, weight):
    idx5 = inputs.T.reshape(NUNITS * NSUB, 128)
    p_out = _gather_scaled(idx5, weight)
    out = (p_out.reshape(NP, DIM // 8, NS // 128, 8, 128)
           .transpose(2, 4, 0, 1, 3)
           .reshape(NS, NP, DIM))
    return out
